# baseline (speedup 1.0000x reference)
import functools

import jax
import jax.numpy as jnp
from jax import lax
from jax.experimental import pallas as pl
from jax.experimental.pallas import tpu as pltpu
from jax.experimental.pallas import tpu_sc as plsc

N = 10000
E = 320000
F = 128

NC = 2
NS = 16
NW = NC * NS
EPW = E // NW
C = 128
NCHF = EPW // C
TAIL = EPW - NCHF * C
NPAIR = NCHF // 2
NP = 10112
RPT = NP // NS
KI = RPT // C
KR = RPT - KI * C
L = 16

_mesh = plsc.VectorSubcoreMesh(core_axis_name="c", subcore_axis_name="s")


@functools.partial(
    pl.kernel,
    out_type=[
        jax.ShapeDtypeStruct((NC, NP, F), jnp.float32),
        jax.ShapeDtypeStruct((NC, NP, F), jnp.float32),
    ],
    mesh=_mesh,
    scratch_types=[
        pltpu.VMEM((C,), jnp.int32),
        pltpu.VMEM((C,), jnp.int32),
        pltpu.VMEM((C,), jnp.int32),
        pltpu.VMEM((C,), jnp.int32),
        pltpu.VMEM((C, F), jnp.float32),
        pltpu.VMEM((C, F), jnp.float32),
        pltpu.VMEM((TAIL,), jnp.int32),
        pltpu.VMEM((TAIL,), jnp.int32),
        pltpu.VMEM_SHARED((NP, F), jnp.float32),
        pltpu.SemaphoreType.DMA,
        pltpu.SemaphoreType.DMA,
        pltpu.SemaphoreType.DMA,
        pltpu.SemaphoreType.DMA,
    ],
)
def _scatter_gather_kernel(src_hbm, dst_hbm, x_hbm, acc_out, cnt_out,
                           src0, src1, dst0, dst1, rows0, rows1, srcT,
                           dstT, acc_s, semg0, semg1, semi0, semi1):
    cid = lax.axis_index("c")
    sid = lax.axis_index("s")
    wid = cid * NS + sid
    base = wid * EPW
    r0 = sid * RPT

    zf16 = jnp.zeros((L,), jnp.float32)
    one16 = jnp.ones((L,), jnp.float32)

    def fill_rows0(vec):
        def fill_body(r, carry):
            for cvec in range(F // L):
                rows0[r, pl.ds(cvec * L, L)] = vec
            return carry

        lax.fori_loop(0, C, fill_body, 0)

    def zero_acc():
        for k in range(KI):
            pltpu.sync_copy(rows0, acc_s.at[pl.ds(r0 + k * C, C)])
        pltpu.sync_copy(rows0.at[pl.ds(0, KR)],
                        acc_s.at[pl.ds(r0 + KI * C, KR)])

    def writeback(out_ref):
        for k in range(KI):
            pltpu.sync_copy(acc_s.at[pl.ds(r0 + k * C, C)], rows1)
            pltpu.sync_copy(rows1, out_ref.at[cid].at[pl.ds(r0 + k * C, C)])
        pltpu.sync_copy(acc_s.at[pl.ds(r0 + KI * C, KR)],
                        rows1.at[pl.ds(0, KR)])
        pltpu.sync_copy(rows1.at[pl.ds(0, KR)],
                        out_ref.at[cid].at[pl.ds(r0 + KI * C, KR)])

    def fetch(j, sbuf, dbuf, sem):
        pltpu.async_copy(src_hbm.at[pl.ds(base + j * C, C)], sbuf, sem)
        pltpu.async_copy(dst_hbm.at[pl.ds(base + j * C, C)], dbuf, sem)

    def fetch_wait(j, sbuf, dbuf, sem):
        pltpu.make_async_copy(src_hbm.at[pl.ds(base + j * C, C)], sbuf,
                              sem).wait()
        pltpu.make_async_copy(dst_hbm.at[pl.ds(base + j * C, C)], dbuf,
                              sem).wait()

    def fetch_d(j, dbuf, sem):
        pltpu.async_copy(dst_hbm.at[pl.ds(base + j * C, C)], dbuf, sem)

    def fetch_d_wait(j, dbuf, sem):
        pltpu.make_async_copy(dst_hbm.at[pl.ds(base + j * C, C)], dbuf,
                              sem).wait()

    fill_rows0(zf16)
    zero_acc()
    fetch(0, src0, dst0, semi0)
    fetch(1, src1, dst1, semi1)
    fetch_wait(0, src0, dst0, semi0)
    pltpu.async_copy(x_hbm.at[src0], rows0, semg0)
    fetch_wait(1, src1, dst1, semi1)
    plsc.subcore_barrier()

    def sum_body(i, carry):
        j0 = 2 * i
        pltpu.async_copy(x_hbm.at[src1], rows1, semg1)
        pltpu.make_async_copy(x_hbm.at[src0], rows0, semg0).wait()
        pltpu.sync_copy(rows0, acc_s.at[dst0], add=True)

        @pl.when(i + 1 < NPAIR)
        def _():
            fetch(j0 + 2, src0, dst0, semi0)

        pltpu.make_async_copy(x_hbm.at[src1], rows1, semg1).wait()
        pltpu.sync_copy(rows1, acc_s.at[dst1], add=True)

        @pl.when(i + 1 < NPAIR)
        def _():
            fetch(j0 + 3, src1, dst1, semi1)
            fetch_wait(j0 + 2, src0, dst0, semi0)
            pltpu.async_copy(x_hbm.at[src0], rows0, semg0)
            fetch_wait(j0 + 3, src1, dst1, semi1)

        return carry

    lax.fori_loop(0, NPAIR, sum_body, 0)

    tb = base + NCHF * C
    pltpu.sync_copy(src_hbm.at[pl.ds(tb, TAIL)], srcT)
    pltpu.sync_copy(dst_hbm.at[pl.ds(tb, TAIL)], dstT)
    pltpu.async_copy(x_hbm.at[srcT], rows0.at[pl.ds(0, TAIL)],
                     semg0).wait()
    pltpu.sync_copy(rows0.at[pl.ds(0, TAIL)], acc_s.at[dstT], add=True)

    fetch_d(0, dst0, semi0)
    fetch_d(1, dst1, semi1)
    fill_rows0(one16)
    plsc.subcore_barrier()
    writeback(acc_out)
    plsc.subcore_barrier()

    fetch_d_wait(0, dst0, semi0)
    pltpu.async_copy(rows0, acc_s.at[dst0], semg0, add=True)
    fetch_d_wait(1, dst1, semi1)

    def cnt_body(i, carry):
        j0 = 2 * i
        pltpu.async_copy(rows0, acc_s.at[dst1], semg1, add=True)
        pltpu.make_async_copy(rows0, acc_s.at[dst0], semg0).wait()

        @pl.when(i + 1 < NPAIR)
        def _():
            fetch_d(j0 + 2, dst0, semi0)

        pltpu.make_async_copy(rows0, acc_s.at[dst1], semg1).wait()

        @pl.when(i + 1 < NPAIR)
        def _():
            fetch_d(j0 + 3, dst1, semi1)
            fetch_d_wait(j0 + 2, dst0, semi0)
            pltpu.async_copy(rows0, acc_s.at[dst0], semg0, add=True)
            fetch_d_wait(j0 + 3, dst1, semi1)

        return carry

    lax.fori_loop(0, NPAIR, cnt_body, 0)

    pltpu.sync_copy(dst_hbm.at[pl.ds(tb, TAIL)], dstT)
    pltpu.sync_copy(rows0.at[pl.ds(0, TAIL)], acc_s.at[dstT], add=True)

    plsc.subcore_barrier()
    writeback(cnt_out)


_BN = 2000


def _combine_body(acc_ref, cnt_ref, out_ref):
    s = acc_ref[0] + acc_ref[1]
    c = (cnt_ref[0, :, 0:1] + cnt_ref[1, :, 0:1]) - s[:, 0:1]
    out_ref[...] = s / jnp.maximum(c, 1.0)


def _combine(acc2, cnt2):
    return pl.pallas_call(
        _combine_body,
        grid=(N // _BN,),
        in_specs=[
            pl.BlockSpec((NC, _BN, F), lambda i: (0, i, 0)),
            pl.BlockSpec((NC, _BN, F), lambda i: (0, i, 0)),
        ],
        out_specs=pl.BlockSpec((_BN, F), lambda i: (i, 0)),
        out_shape=jax.ShapeDtypeStruct((N, F), jnp.float32),
    )(acc2, cnt2)


def kernel(x, edge_index):
    src = edge_index[0]
    dst = edge_index[1]
    acc2, cnt2 = _scatter_gather_kernel(src, dst, x)
    return _combine(acc2, cnt2)


# speedup vs baseline: 1.0285x; 1.0285x over previous
import functools

import jax
import jax.numpy as jnp
from jax import lax
from jax.experimental import pallas as pl
from jax.experimental.pallas import tpu as pltpu
from jax.experimental.pallas import tpu_sc as plsc

N = 10000
E = 320000
F = 128

NC = 2
NS = 16
NW = NC * NS
EPW = E // NW
C = 128
NCHF = EPW // C
TAIL = EPW - NCHF * C
NPAIR = NCHF // 2
NP = 10112
RPT = NP // NS
KI = RPT // C
KR = RPT - KI * C
L = 16

_mesh = plsc.VectorSubcoreMesh(core_axis_name="c", subcore_axis_name="s")


@functools.partial(
    pl.kernel,
    out_type=[
        jax.ShapeDtypeStruct((NC, NP, F), jnp.float32),
        jax.ShapeDtypeStruct((NC, NP, F), jnp.float32),
    ],
    mesh=_mesh,
    scratch_types=[
        pltpu.VMEM((C,), jnp.int32),
        pltpu.VMEM((C,), jnp.int32),
        pltpu.VMEM((C,), jnp.int32),
        pltpu.VMEM((C,), jnp.int32),
        pltpu.VMEM((C, F), jnp.float32),
        pltpu.VMEM((C, F), jnp.float32),
        pltpu.VMEM((TAIL,), jnp.int32),
        pltpu.VMEM((TAIL,), jnp.int32),
        pltpu.VMEM_SHARED((NP, F), jnp.float32),
        pltpu.SemaphoreType.DMA,
        pltpu.SemaphoreType.DMA,
        pltpu.SemaphoreType.DMA,
        pltpu.SemaphoreType.DMA,
    ],
)
def _scatter_gather_kernel(src_hbm, dst_hbm, x_hbm, acc_out, cnt_out,
                           src0, src1, dst0, dst1, rows0, rows1, srcT,
                           dstT, acc_s, semg0, semg1, semi0, semi1):
    cid = lax.axis_index("c")
    sid = lax.axis_index("s")
    wid = cid * NS + sid
    base = wid * EPW
    r0 = sid * RPT

    zf16 = jnp.zeros((L,), jnp.float32)
    one16 = jnp.ones((L,), jnp.float32)

    def fill_rows0(vec):
        def fill_body(r, carry):
            for cvec in range(F // L):
                rows0[r, pl.ds(cvec * L, L)] = vec
            return carry

        lax.fori_loop(0, C, fill_body, 0)

    def zero_acc():
        for k in range(KI):
            pltpu.sync_copy(rows0, acc_s.at[pl.ds(r0 + k * C, C)])
        pltpu.sync_copy(rows0.at[pl.ds(0, KR)],
                        acc_s.at[pl.ds(r0 + KI * C, KR)])

    def writeback(out_ref):
        for k in range(KI):
            pltpu.sync_copy(acc_s.at[pl.ds(r0 + k * C, C)], rows1)
            pltpu.sync_copy(rows1, out_ref.at[cid].at[pl.ds(r0 + k * C, C)])
        pltpu.sync_copy(acc_s.at[pl.ds(r0 + KI * C, KR)],
                        rows1.at[pl.ds(0, KR)])
        pltpu.sync_copy(rows1.at[pl.ds(0, KR)],
                        out_ref.at[cid].at[pl.ds(r0 + KI * C, KR)])

    def fetch(j, sbuf, dbuf, sem):
        pltpu.async_copy(src_hbm.at[pl.ds(base + j * C, C)], sbuf, sem)
        pltpu.async_copy(dst_hbm.at[pl.ds(base + j * C, C)], dbuf, sem)

    def fetch_wait(j, sbuf, dbuf, sem):
        pltpu.make_async_copy(src_hbm.at[pl.ds(base + j * C, C)], sbuf,
                              sem).wait()
        pltpu.make_async_copy(dst_hbm.at[pl.ds(base + j * C, C)], dbuf,
                              sem).wait()

    def fetch_d(j, dbuf, sem):
        pltpu.async_copy(dst_hbm.at[pl.ds(base + j * C, C)], dbuf, sem)

    def fetch_d_wait(j, dbuf, sem):
        pltpu.make_async_copy(dst_hbm.at[pl.ds(base + j * C, C)], dbuf,
                              sem).wait()

    fill_rows0(zf16)
    zero_acc()
    plsc.subcore_barrier()

    fetch(0, src0, dst0, semi0)
    fetch(1, src1, dst1, semi1)
    fetch_wait(0, src0, dst0, semi0)
    pltpu.async_copy(x_hbm.at[src0], rows0, semg0)
    fetch_wait(1, src1, dst1, semi1)

    def sum_body(i, carry):
        j0 = 2 * i
        pltpu.async_copy(x_hbm.at[src1], rows1, semg1)
        pltpu.make_async_copy(x_hbm.at[src0], rows0, semg0).wait()
        pltpu.sync_copy(rows0, acc_s.at[dst0], add=True)

        @pl.when(i + 1 < NPAIR)
        def _():
            fetch(j0 + 2, src0, dst0, semi0)

        pltpu.make_async_copy(x_hbm.at[src1], rows1, semg1).wait()
        pltpu.sync_copy(rows1, acc_s.at[dst1], add=True)

        @pl.when(i + 1 < NPAIR)
        def _():
            fetch(j0 + 3, src1, dst1, semi1)
            fetch_wait(j0 + 2, src0, dst0, semi0)
            pltpu.async_copy(x_hbm.at[src0], rows0, semg0)
            fetch_wait(j0 + 3, src1, dst1, semi1)

        return carry

    lax.fori_loop(0, NPAIR, sum_body, 0)

    tb = base + NCHF * C
    pltpu.sync_copy(src_hbm.at[pl.ds(tb, TAIL)], srcT)
    pltpu.sync_copy(dst_hbm.at[pl.ds(tb, TAIL)], dstT)
    pltpu.async_copy(x_hbm.at[srcT], rows0.at[pl.ds(0, TAIL)],
                     semg0).wait()
    pltpu.sync_copy(rows0.at[pl.ds(0, TAIL)], acc_s.at[dstT], add=True)

    plsc.subcore_barrier()
    writeback(acc_out)
    plsc.subcore_barrier()

    fill_rows0(one16)
    pltpu.sync_copy(dst_hbm.at[pl.ds(base, C)], dst0)
    pltpu.async_copy(rows0, acc_s.at[dst0], semg0, add=True)
    pltpu.sync_copy(dst_hbm.at[pl.ds(base + C, C)], dst1)

    def cnt_body(i, carry):
        j0 = 2 * i
        pltpu.async_copy(rows0, acc_s.at[dst1], semg1, add=True)
        pltpu.make_async_copy(rows0, acc_s.at[dst0], semg0).wait()

        @pl.when(i + 1 < NPAIR)
        def _():
            fetch_d(j0 + 2, dst0, semi0)

        pltpu.make_async_copy(rows0, acc_s.at[dst1], semg1).wait()

        @pl.when(i + 1 < NPAIR)
        def _():
            fetch_d(j0 + 3, dst1, semi1)
            fetch_d_wait(j0 + 2, dst0, semi0)
            pltpu.async_copy(rows0, acc_s.at[dst0], semg0, add=True)
            fetch_d_wait(j0 + 3, dst1, semi1)

        return carry

    lax.fori_loop(0, NPAIR, cnt_body, 0)

    pltpu.sync_copy(dst_hbm.at[pl.ds(tb, TAIL)], dstT)
    pltpu.sync_copy(rows0.at[pl.ds(0, TAIL)], acc_s.at[dstT], add=True)

    plsc.subcore_barrier()
    writeback(cnt_out)


_BN = 2000


def _combine_body(acc_ref, cnt_ref, out_ref):
    s = acc_ref[0] + acc_ref[1]
    c = (cnt_ref[0, :, 0:1] + cnt_ref[1, :, 0:1]) - s[:, 0:1]
    out_ref[...] = s / jnp.maximum(c, 1.0)


def _combine(acc2, cnt2):
    return pl.pallas_call(
        _combine_body,
        grid=(N // _BN,),
        in_specs=[
            pl.BlockSpec((NC, _BN, F), lambda i: (0, i, 0)),
            pl.BlockSpec((NC, _BN, F), lambda i: (0, i, 0)),
        ],
        out_specs=pl.BlockSpec((_BN, F), lambda i: (i, 0)),
        out_shape=jax.ShapeDtypeStruct((N, F), jnp.float32),
    )(acc2, cnt2)


def kernel(x, edge_index):
    src = edge_index[0]
    dst = edge_index[1]
    acc2, cnt2 = _scatter_gather_kernel(src, dst, x)
    return _combine(acc2, cnt2)


# speedup vs baseline: 1.0798x; 1.0498x over previous
import functools

import jax
import jax.numpy as jnp
from jax import lax
from jax.experimental import pallas as pl
from jax.experimental.pallas import tpu as pltpu
from jax.experimental.pallas import tpu_sc as plsc

N = 10000
E = 320000
F = 128

NC = 2
NS = 16
NW = NC * NS
EPW = E // NW
C = 128
NCHF = EPW // C
TAIL = EPW - NCHF * C
NPAIR = NCHF // 2
NP = 10112
RPT = NP // NS
KI = RPT // C
KR = RPT - KI * C
L = 16

_mesh = plsc.VectorSubcoreMesh(core_axis_name="c", subcore_axis_name="s")


@functools.partial(
    pl.kernel,
    out_type=[
        jax.ShapeDtypeStruct((NC, NP, F), jnp.float32),
        jax.ShapeDtypeStruct((NC, NP, F), jnp.float32),
    ],
    mesh=_mesh,
    scratch_types=[
        pltpu.VMEM((C,), jnp.int32),
        pltpu.VMEM((C,), jnp.int32),
        pltpu.VMEM((C,), jnp.int32),
        pltpu.VMEM((C,), jnp.int32),
        pltpu.VMEM((C, F), jnp.float32),
        pltpu.VMEM((C, F), jnp.float32),
        pltpu.VMEM((TAIL,), jnp.int32),
        pltpu.VMEM((TAIL,), jnp.int32),
        pltpu.VMEM_SHARED((NP, F), jnp.float32),
        pltpu.SemaphoreType.DMA,
        pltpu.SemaphoreType.DMA,
        pltpu.SemaphoreType.DMA,
        pltpu.SemaphoreType.DMA,
    ],
)
def _scatter_gather_kernel(ei_hbm, x_hbm, acc_out, cnt_out,
                           src0, src1, dst0, dst1, rows0, rows1, srcT,
                           dstT, acc_s, semg0, semg1, semi0, semi1):
    cid = lax.axis_index("c")
    sid = lax.axis_index("s")
    wid = cid * NS + sid
    base = wid * EPW
    r0 = sid * RPT

    zf16 = jnp.zeros((L,), jnp.float32)
    one16 = jnp.ones((L,), jnp.float32)

    def fill_rows0(vec):
        def fill_body(r, carry):
            for cvec in range(F // L):
                rows0[r, pl.ds(cvec * L, L)] = vec
            return carry

        lax.fori_loop(0, C, fill_body, 0)

    def zero_acc():
        for k in range(KI):
            pltpu.sync_copy(rows0, acc_s.at[pl.ds(r0 + k * C, C)])
        pltpu.sync_copy(rows0.at[pl.ds(0, KR)],
                        acc_s.at[pl.ds(r0 + KI * C, KR)])

    def writeback(out_ref):
        for k in range(KI):
            pltpu.sync_copy(acc_s.at[pl.ds(r0 + k * C, C)], rows1)
            pltpu.sync_copy(rows1, out_ref.at[cid].at[pl.ds(r0 + k * C, C)])
        pltpu.sync_copy(acc_s.at[pl.ds(r0 + KI * C, KR)],
                        rows1.at[pl.ds(0, KR)])
        pltpu.sync_copy(rows1.at[pl.ds(0, KR)],
                        out_ref.at[cid].at[pl.ds(r0 + KI * C, KR)])

    def fetch(j, sbuf, dbuf, sem):
        pltpu.async_copy(ei_hbm.at[pl.ds(base + j * C, C)], sbuf, sem)
        pltpu.async_copy(ei_hbm.at[pl.ds(E + base + j * C, C)], dbuf, sem)

    def fetch_wait(j, sbuf, dbuf, sem):
        pltpu.make_async_copy(ei_hbm.at[pl.ds(base + j * C, C)], sbuf,
                              sem).wait()
        pltpu.make_async_copy(ei_hbm.at[pl.ds(E + base + j * C, C)], dbuf,
                              sem).wait()

    def fetch_d(j, dbuf, sem):
        pltpu.async_copy(ei_hbm.at[pl.ds(E + base + j * C, C)], dbuf, sem)

    def fetch_d_wait(j, dbuf, sem):
        pltpu.make_async_copy(ei_hbm.at[pl.ds(E + base + j * C, C)], dbuf,
                              sem).wait()

    fill_rows0(zf16)
    zero_acc()
    plsc.subcore_barrier()

    fetch(0, src0, dst0, semi0)
    fetch(1, src1, dst1, semi1)
    fetch_wait(0, src0, dst0, semi0)
    pltpu.async_copy(x_hbm.at[src0], rows0, semg0)
    fetch_wait(1, src1, dst1, semi1)

    def sum_body(i, carry):
        j0 = 2 * i
        pltpu.async_copy(x_hbm.at[src1], rows1, semg1)
        pltpu.make_async_copy(x_hbm.at[src0], rows0, semg0).wait()
        pltpu.sync_copy(rows0, acc_s.at[dst0], add=True)

        @pl.when(i + 1 < NPAIR)
        def _():
            fetch(j0 + 2, src0, dst0, semi0)

        pltpu.make_async_copy(x_hbm.at[src1], rows1, semg1).wait()
        pltpu.sync_copy(rows1, acc_s.at[dst1], add=True)

        @pl.when(i + 1 < NPAIR)
        def _():
            fetch(j0 + 3, src1, dst1, semi1)
            fetch_wait(j0 + 2, src0, dst0, semi0)
            pltpu.async_copy(x_hbm.at[src0], rows0, semg0)
            fetch_wait(j0 + 3, src1, dst1, semi1)

        return carry

    lax.fori_loop(0, NPAIR, sum_body, 0)

    tb = base + NCHF * C
    pltpu.sync_copy(ei_hbm.at[pl.ds(tb, TAIL)], srcT)
    pltpu.sync_copy(ei_hbm.at[pl.ds(E + tb, TAIL)], dstT)
    pltpu.async_copy(x_hbm.at[srcT], rows0.at[pl.ds(0, TAIL)],
                     semg0).wait()
    pltpu.sync_copy(rows0.at[pl.ds(0, TAIL)], acc_s.at[dstT], add=True)

    plsc.subcore_barrier()
    writeback(acc_out)
    plsc.subcore_barrier()

    fill_rows0(one16)
    pltpu.sync_copy(ei_hbm.at[pl.ds(E + base, C)], dst0)
    pltpu.async_copy(rows0, acc_s.at[dst0], semg0, add=True)
    pltpu.sync_copy(ei_hbm.at[pl.ds(E + base + C, C)], dst1)

    def cnt_body(i, carry):
        j0 = 2 * i
        pltpu.async_copy(rows0, acc_s.at[dst1], semg1, add=True)
        pltpu.make_async_copy(rows0, acc_s.at[dst0], semg0).wait()

        @pl.when(i + 1 < NPAIR)
        def _():
            fetch_d(j0 + 2, dst0, semi0)

        pltpu.make_async_copy(rows0, acc_s.at[dst1], semg1).wait()

        @pl.when(i + 1 < NPAIR)
        def _():
            fetch_d(j0 + 3, dst1, semi1)
            fetch_d_wait(j0 + 2, dst0, semi0)
            pltpu.async_copy(rows0, acc_s.at[dst0], semg0, add=True)
            fetch_d_wait(j0 + 3, dst1, semi1)

        return carry

    lax.fori_loop(0, NPAIR, cnt_body, 0)

    pltpu.sync_copy(ei_hbm.at[pl.ds(E + tb, TAIL)], dstT)
    pltpu.sync_copy(rows0.at[pl.ds(0, TAIL)], acc_s.at[dstT], add=True)

    plsc.subcore_barrier()
    writeback(cnt_out)


_BN = 2000


def _combine_body(acc_ref, cnt_ref, out_ref):
    s = acc_ref[0] + acc_ref[1]
    c = (cnt_ref[0, :, 0:1] + cnt_ref[1, :, 0:1]) - s[:, 0:1]
    out_ref[...] = s / jnp.maximum(c, 1.0)


def _combine(acc2, cnt2):
    return pl.pallas_call(
        _combine_body,
        grid=(N // _BN,),
        in_specs=[
            pl.BlockSpec((NC, _BN, F), lambda i: (0, i, 0)),
            pl.BlockSpec((NC, _BN, F), lambda i: (0, i, 0)),
        ],
        out_specs=pl.BlockSpec((_BN, F), lambda i: (i, 0)),
        out_shape=jax.ShapeDtypeStruct((N, F), jnp.float32),
    )(acc2, cnt2)


def kernel(x, edge_index):
    acc2, cnt2 = _scatter_gather_kernel(edge_index.reshape(2 * E), x)
    return _combine(acc2, cnt2)
